# Initial kernel scaffold; baseline (speedup 1.0000x reference)
import jax
import jax.numpy as jnp
from jax.experimental import pallas as pl


def kernel(x, edge_index, edge_attr, u, batch, W1, b1, W2, b2, gamma, beta):
    raise NotImplementedError("write your pallas kernel here")


# speedup vs baseline: 13.1381x; 13.1381x over previous
import functools

import jax
import jax.numpy as jnp
from jax import lax
from jax.experimental import pallas as pl
from jax.experimental.pallas import tpu as pltpu
from jax.experimental.pallas import tpu_sc as plsc

N = 100000
E = 1600000
D_NODE = 128
D_EDGE = 16
D_GLOB = 64
B = 256

NC = 2
NS = 16
NW = NC * NS
L = 16

U = 128
EU = E // U
NU = N // U
N_TAIL = N - NU * U
K_E = (EU + NW - 1) // NW
K_N = (NU + NW - 1) // NW


def _sc_body(src_hbm, attr_hbm, x_hbm, batch_hbm, ze_hbm, zn_hbm, zf_hbm,
             esum_hbm, nsum_hbm, ecnt_hbm, ncnt_hbm,
             batch_spm, eacc_spm, nacc_spm,
             idx_v, seg_v, attr_v, x_v, nb_v, hist_e, hist_n, cnt_v,
             nb_tail, x_tail, sem):
    c = lax.axis_index("c")
    s = lax.axis_index("s")
    wid = s * NC + c

    @pl.when(s == 0)
    def _():
        pltpu.sync_copy(batch_hbm, batch_spm)

    pltpu.sync_copy(ze_hbm, eacc_spm.at[pl.ds(s * (B // NS), B // NS)])
    pltpu.sync_copy(zn_hbm, nacc_spm.at[pl.ds(s * (B // NS), B // NS)])
    pltpu.sync_copy(zf_hbm, hist_e)
    pltpu.sync_copy(zf_hbm, hist_n)
    plsc.subcore_barrier()

    lane = lax.broadcasted_iota(jnp.int32, (L,), 0) * B
    ones = jnp.ones((L,), jnp.float32)

    def edge_step(k, carry):
        t = k * NW + wid

        @pl.when(t < EU)
        def _():
            base = t * U
            pltpu.sync_copy(src_hbm.at[pl.ds(base, U)], idx_v)
            pltpu.async_copy(batch_spm.at[idx_v], seg_v, sem).wait()
            pltpu.sync_copy(attr_hbm.at[pl.ds(base, U)], attr_v)
            pltpu.sync_copy(attr_v, eacc_spm.at[seg_v], add=True)
            for i in range(U // L):
                segs = seg_v[pl.ds(i * L, L)]
                plsc.addupdate_scatter(hist_e, [lane + segs], ones)

        return carry

    lax.fori_loop(0, K_E, edge_step, 0)

    def node_step(k, carry):
        t = k * NW + wid

        @pl.when(t < NU)
        def _():
            base = t * U
            pltpu.sync_copy(batch_hbm.at[pl.ds(base, U)], nb_v)
            pltpu.sync_copy(x_hbm.at[pl.ds(base, U)], x_v)
            pltpu.sync_copy(x_v, nacc_spm.at[nb_v], add=True)
            for i in range(U // L):
                segs = nb_v[pl.ds(i * L, L)]
                plsc.addupdate_scatter(hist_n, [lane + segs], ones)

        return carry

    lax.fori_loop(0, K_N, node_step, 0)

    @pl.when(wid == 0)
    def _():
        base = NU * U
        pltpu.sync_copy(batch_hbm.at[pl.ds(base, N_TAIL)], nb_tail)
        pltpu.sync_copy(x_hbm.at[pl.ds(base, N_TAIL)], x_tail)
        pltpu.sync_copy(x_tail, nacc_spm.at[nb_tail], add=True)
        for i in range(N_TAIL // L):
            segs = nb_tail[pl.ds(i * L, L)]
            plsc.addupdate_scatter(hist_n, [lane + segs], ones)

    for hist, out in ((hist_e, ecnt_hbm), (hist_n, ncnt_hbm)):
        for b in range(B // L):
            acc = hist[pl.ds(b * L, L)]
            for l in range(1, L):
                acc = acc + hist[pl.ds(l * B + b * L, L)]
            cnt_v[pl.ds(b * L, L)] = acc
        pltpu.sync_copy(cnt_v, out.at[wid])

    plsc.subcore_barrier()

    @pl.when(s == 0)
    def _():
        pltpu.sync_copy(eacc_spm, esum_hbm.at[c])
        pltpu.sync_copy(nacc_spm, nsum_hbm.at[c])


def _sc_segment_sums(edge_src, edge_attr, x, batch):
    mesh = plsc.VectorSubcoreMesh(core_axis_name="c", subcore_axis_name="s",
                                  num_cores=NC, num_subcores=NS)
    ze = jnp.zeros((B // NS, D_EDGE), jnp.float32)
    zn = jnp.zeros((B // NS, D_NODE), jnp.float32)
    zf = jnp.zeros((L * B,), jnp.float32)
    out_type = (
        jax.ShapeDtypeStruct((NC, B, D_EDGE), jnp.float32),
        jax.ShapeDtypeStruct((NC, B, D_NODE), jnp.float32),
        jax.ShapeDtypeStruct((NW, B), jnp.float32),
        jax.ShapeDtypeStruct((NW, B), jnp.float32),
    )
    scratch = [
        pltpu.VMEM_SHARED((N,), jnp.int32),
        pltpu.VMEM_SHARED((B, D_EDGE), jnp.float32),
        pltpu.VMEM_SHARED((B, D_NODE), jnp.float32),
        pltpu.VMEM((U,), jnp.int32),
        pltpu.VMEM((U,), jnp.int32),
        pltpu.VMEM((U, D_EDGE), jnp.float32),
        pltpu.VMEM((U, D_NODE), jnp.float32),
        pltpu.VMEM((U,), jnp.int32),
        pltpu.VMEM((L * B,), jnp.float32),
        pltpu.VMEM((L * B,), jnp.float32),
        pltpu.VMEM((B,), jnp.float32),
        pltpu.VMEM((N_TAIL,), jnp.int32),
        pltpu.VMEM((N_TAIL, D_NODE), jnp.float32),
        pltpu.SemaphoreType.DMA,
    ]
    fn = pl.kernel(_sc_body, out_type=out_type, mesh=mesh,
                   scratch_types=scratch,
                   compiler_params=pltpu.CompilerParams(
                       needs_layout_passes=False))
    return fn(edge_src, edge_attr, x, batch, ze, zn, zf)


def _tc_body(u_ref, esum_ref, nsum_ref, ecnt_ref, ncnt_ref,
             w1u_ref, w1e_ref, w1n_ref, b1_ref, w2_ref, b2_ref,
             gamma_ref, beta_ref, out_ref):
    e_sum = esum_ref[0] + esum_ref[1]
    n_sum = nsum_ref[0] + nsum_ref[1]
    e_cnt = jnp.sum(ecnt_ref[...], axis=0)
    n_cnt = jnp.sum(ncnt_ref[...], axis=0)
    e_mean = e_sum / jnp.maximum(e_cnt, 1.0)[:, None]
    n_mean = n_sum / jnp.maximum(n_cnt, 1.0)[:, None]
    u = u_ref[...]
    h = (jnp.dot(u, w1u_ref[...], preferred_element_type=jnp.float32)
         + jnp.dot(e_mean, w1e_ref[...], preferred_element_type=jnp.float32)
         + jnp.dot(n_mean, w1n_ref[...], preferred_element_type=jnp.float32)
         + b1_ref[...])
    h = jnp.maximum(h, 0.0)
    y = jnp.dot(h, w2_ref[...], preferred_element_type=jnp.float32) + b2_ref[...] + u
    mu = jnp.mean(y, axis=-1, keepdims=True)
    var = jnp.mean((y - mu) ** 2, axis=-1, keepdims=True)
    y = (y - mu) * lax.rsqrt(var + 1e-5)
    out_ref[...] = y * gamma_ref[...] + beta_ref[...]


def kernel(x, edge_index, edge_attr, u, batch, W1, b1, W2, b2, gamma, beta):
    edge_src = edge_index[0]
    esum, nsum, ecnt, ncnt = _sc_segment_sums(edge_src, edge_attr, x, batch)
    w1u = W1[:D_GLOB]
    w1e = W1[D_GLOB:D_GLOB + D_EDGE]
    w1n = W1[D_GLOB + D_EDGE:]
    return pl.pallas_call(
        _tc_body,
        out_shape=jax.ShapeDtypeStruct((B, D_GLOB), jnp.float32),
    )(u, esum, nsum, ecnt, ncnt, w1u, w1e, w1n,
      b1[None, :], W2, b2[None, :], gamma[None, :], beta[None, :])


# speedup vs baseline: 22.0113x; 1.6754x over previous
import functools

import jax
import jax.numpy as jnp
from jax import lax
from jax.experimental import pallas as pl
from jax.experimental.pallas import tpu as pltpu
from jax.experimental.pallas import tpu_sc as plsc

N = 100000
E = 1600000
D_NODE = 128
D_EDGE = 16
D_GLOB = 64
B = 256

NC = 2
NS = 16
NW = NC * NS
L = 16

UE = 2000
EU = E // UE
K_E = (EU + NW - 1) // NW
UN = 400
NU = N // UN
K_N = (NU + NW - 1) // NW


def _sc_body(src_hbm, attr_hbm, x_hbm, batch_hbm, ze_hbm, zn_hbm, zf_hbm,
             esum_hbm, nsum_hbm, ecnt_hbm, ncnt_hbm,
             batch_spm, eacc_spm, nacc_spm,
             idx_v, seg_v, attr_v, x_v, nb_v, hist_e, hist_n, cnt_v, sem):
    c = lax.axis_index("c")
    s = lax.axis_index("s")
    wid = s * NC + c

    @pl.when(s == 0)
    def _():
        pltpu.sync_copy(batch_hbm, batch_spm)

    pltpu.sync_copy(ze_hbm, eacc_spm.at[pl.ds(s * (B // NS), B // NS)])
    pltpu.sync_copy(zn_hbm, nacc_spm.at[pl.ds(s * (B // NS), B // NS)])
    pltpu.sync_copy(zf_hbm, hist_e)
    pltpu.sync_copy(zf_hbm, hist_n)
    plsc.subcore_barrier()

    lane = lax.broadcasted_iota(jnp.int32, (L,), 0) * B
    ones = jnp.ones((L,), jnp.float32)

    def edge_step(k, carry):
        t = k * NW + wid

        @pl.when(t < EU)
        def _():
            base = pl.multiple_of(t * UE, UE)
            pltpu.sync_copy(src_hbm.at[pl.ds(base, UE)], idx_v)
            pltpu.async_copy(batch_spm.at[idx_v], seg_v, sem).wait()
            pltpu.sync_copy(attr_hbm.at[pl.ds(base, UE)], attr_v)
            pltpu.sync_copy(attr_v, eacc_spm.at[seg_v], add=True)
            for i in range(UE // L):
                segs = seg_v[pl.ds(i * L, L)]
                plsc.addupdate_scatter(hist_e, [lane + segs], ones)

        return carry

    lax.fori_loop(0, K_E, edge_step, 0)

    def node_step(k, carry):
        t = k * NW + wid

        @pl.when(t < NU)
        def _():
            base = pl.multiple_of(t * UN, UN)
            pltpu.sync_copy(batch_hbm.at[pl.ds(base, UN)], nb_v)
            pltpu.sync_copy(x_hbm.at[pl.ds(base, UN)], x_v)
            pltpu.sync_copy(x_v, nacc_spm.at[nb_v], add=True)
            for i in range(UN // L):
                segs = nb_v[pl.ds(i * L, L)]
                plsc.addupdate_scatter(hist_n, [lane + segs], ones)

        return carry

    lax.fori_loop(0, K_N, node_step, 0)

    for hist, out in ((hist_e, ecnt_hbm), (hist_n, ncnt_hbm)):
        for b in range(B // L):
            acc = hist[pl.ds(b * L, L)]
            for l in range(1, L):
                acc = acc + hist[pl.ds(l * B + b * L, L)]
            cnt_v[pl.ds(b * L, L)] = acc
        pltpu.sync_copy(cnt_v, out.at[wid])

    plsc.subcore_barrier()

    @pl.when(s == 0)
    def _():
        pltpu.sync_copy(eacc_spm, esum_hbm.at[c])
        pltpu.sync_copy(nacc_spm, nsum_hbm.at[c])


def _sc_segment_sums(edge_src, edge_attr, x, batch):
    mesh = plsc.VectorSubcoreMesh(core_axis_name="c", subcore_axis_name="s",
                                  num_cores=NC, num_subcores=NS)
    ze = jnp.zeros((B // NS, D_EDGE), jnp.float32)
    zn = jnp.zeros((B // NS, D_NODE), jnp.float32)
    zf = jnp.zeros((L * B,), jnp.float32)
    out_type = (
        jax.ShapeDtypeStruct((NC, B, D_EDGE), jnp.float32),
        jax.ShapeDtypeStruct((NC, B, D_NODE), jnp.float32),
        jax.ShapeDtypeStruct((NW, B), jnp.float32),
        jax.ShapeDtypeStruct((NW, B), jnp.float32),
    )
    scratch = [
        pltpu.VMEM_SHARED((N,), jnp.int32),
        pltpu.VMEM_SHARED((B, D_EDGE), jnp.float32),
        pltpu.VMEM_SHARED((B, D_NODE), jnp.float32),
        pltpu.VMEM((UE,), jnp.int32),
        pltpu.VMEM((UE,), jnp.int32),
        pltpu.VMEM((UE, D_EDGE), jnp.float32),
        pltpu.VMEM((UN, D_NODE), jnp.float32),
        pltpu.VMEM((UN,), jnp.int32),
        pltpu.VMEM((L * B,), jnp.float32),
        pltpu.VMEM((L * B,), jnp.float32),
        pltpu.VMEM((B,), jnp.float32),
        pltpu.SemaphoreType.DMA,
    ]
    fn = pl.kernel(_sc_body, out_type=out_type, mesh=mesh,
                   scratch_types=scratch,
                   compiler_params=pltpu.CompilerParams(
                       needs_layout_passes=False,
                       use_tc_tiling_on_sc=False))
    return fn(edge_src, edge_attr, x, batch, ze, zn, zf)


def _tc_body(u_ref, esum_ref, nsum_ref, ecnt_ref, ncnt_ref,
             w1u_ref, w1e_ref, w1n_ref, b1_ref, w2_ref, b2_ref,
             gamma_ref, beta_ref, out_ref):
    e_sum = esum_ref[0] + esum_ref[1]
    n_sum = nsum_ref[0] + nsum_ref[1]
    e_cnt = jnp.sum(ecnt_ref[...], axis=0)
    n_cnt = jnp.sum(ncnt_ref[...], axis=0)
    e_mean = e_sum / jnp.maximum(e_cnt, 1.0)[:, None]
    n_mean = n_sum / jnp.maximum(n_cnt, 1.0)[:, None]
    u = u_ref[...]
    h = (jnp.dot(u, w1u_ref[...], preferred_element_type=jnp.float32)
         + jnp.dot(e_mean, w1e_ref[...], preferred_element_type=jnp.float32)
         + jnp.dot(n_mean, w1n_ref[...], preferred_element_type=jnp.float32)
         + b1_ref[...])
    h = jnp.maximum(h, 0.0)
    y = jnp.dot(h, w2_ref[...], preferred_element_type=jnp.float32) + b2_ref[...] + u
    mu = jnp.mean(y, axis=-1, keepdims=True)
    var = jnp.mean((y - mu) ** 2, axis=-1, keepdims=True)
    y = (y - mu) * lax.rsqrt(var + 1e-5)
    out_ref[...] = y * gamma_ref[...] + beta_ref[...]


def kernel(x, edge_index, edge_attr, u, batch, W1, b1, W2, b2, gamma, beta):
    edge_src = edge_index[0]
    esum, nsum, ecnt, ncnt = _sc_segment_sums(edge_src, edge_attr, x, batch)
    w1u = W1[:D_GLOB]
    w1e = W1[D_GLOB:D_GLOB + D_EDGE]
    w1n = W1[D_GLOB + D_EDGE:]
    return pl.pallas_call(
        _tc_body,
        out_shape=jax.ShapeDtypeStruct((B, D_GLOB), jnp.float32),
    )(u, esum, nsum, ecnt, ncnt, w1u, w1e, w1n,
      b1[None, :], W2, b2[None, :], gamma[None, :], beta[None, :])


# speedup vs baseline: 23.9006x; 1.0858x over previous
import functools

import jax
import jax.numpy as jnp
from jax import lax
from jax.experimental import pallas as pl
from jax.experimental.pallas import tpu as pltpu
from jax.experimental.pallas import tpu_sc as plsc

N = 100000
E = 1600000
D_NODE = 128
D_EDGE = 16
D_GLOB = 64
B = 256

NC = 2
NS = 16
NW = NC * NS
L = 16

UE = 2000
EU = E // UE
K_E = EU // NW
UN = 160
NU = N // UN
K_N = (NU + NW - 1) // NW


def _sc_body(src_hbm, attr_hbm, x_hbm, batch_hbm, ze_hbm, zn_hbm, zf_hbm,
             esum_hbm, nsum_hbm, ecnt_hbm, ncnt_hbm,
             batch_spm, eacc_spm, nacc_spm,
             idx0, idx1, seg0, seg1, attr0, attr1, x0, x1, nb0, nb1,
             hist, cnt_v, si0, si1, sa0, sa1, sg0, sg1, ss0, ss1):
    c = lax.axis_index("c")
    s = lax.axis_index("s")
    wid = s * NC + c

    @pl.when(s == 0)
    def _():
        pltpu.sync_copy(batch_hbm, batch_spm)

    pltpu.sync_copy(ze_hbm, eacc_spm.at[pl.ds(s * (B // NS), B // NS)])
    pltpu.sync_copy(zn_hbm, nacc_spm.at[pl.ds(s * (B // NS), B // NS)])
    pltpu.sync_copy(zf_hbm, hist)
    plsc.subcore_barrier()

    lane = lax.broadcasted_iota(jnp.int32, (L,), 0) * B
    ones = jnp.ones((L,), jnp.float32)

    def histo(segb, n):
        for i in range(n // L):
            segs = segb[pl.ds(i * L, L)]
            plsc.addupdate_scatter(hist, [lane + segs], ones)

    def cnt_out(out):
        for b in range(B // L):
            acc = hist[pl.ds(b * L, L)]
            for l in range(1, L):
                acc = acc + hist[pl.ds(l * B + b * L, L)]
            cnt_v[pl.ds(b * L, L)] = acc
        pltpu.sync_copy(cnt_v, out.at[wid])

    def e_loads(t, idxb, attrb, s_i, s_a):
        base = pl.multiple_of(t * UE, UE)
        di = pltpu.async_copy(src_hbm.at[pl.ds(base, UE)], idxb, s_i)
        da = pltpu.async_copy(attr_hbm.at[pl.ds(base, UE)], attrb, s_a)
        return di, da

    def edge_pair(j, carry):
        t0 = (2 * j) * NW + wid
        di0, da0 = e_loads(t0, idx0, attr0, si0, sa0)
        di0.wait()
        dg0 = pltpu.async_copy(batch_spm.at[idx0], seg0, sg0)
        di1, da1 = e_loads(t0 + NW, idx1, attr1, si1, sa1)
        dg0.wait()
        da0.wait()
        ds0 = pltpu.async_copy(attr0, eacc_spm.at[seg0], ss0, add=True)
        histo(seg0, UE)
        di1.wait()
        dg1 = pltpu.async_copy(batch_spm.at[idx1], seg1, sg1)
        dg1.wait()
        da1.wait()
        ds1 = pltpu.async_copy(attr1, eacc_spm.at[seg1], ss1, add=True)
        histo(seg1, UE)
        ds0.wait()
        ds1.wait()
        return carry

    lax.fori_loop(0, K_E // 2, edge_pair, 0)

    t_tail = (K_E - 1) * NW + wid
    di0, da0 = e_loads(t_tail, idx0, attr0, si0, sa0)
    di0.wait()
    pltpu.async_copy(batch_spm.at[idx0], seg0, sg0).wait()
    da0.wait()
    ds0 = pltpu.async_copy(attr0, eacc_spm.at[seg0], ss0, add=True)
    histo(seg0, UE)
    ds0.wait()

    cnt_out(ecnt_hbm)
    pltpu.sync_copy(zf_hbm, hist)

    def n_loads(t, nbb, xb, s_i, s_a):
        base = pl.multiple_of(t * UN, UN)
        dn = pltpu.async_copy(batch_hbm.at[pl.ds(base, UN)], nbb, s_i)
        dx = pltpu.async_copy(x_hbm.at[pl.ds(base, UN)], xb, s_a)
        return dn, dx

    def node_pair(j, carry):
        t0 = (2 * j) * NW + wid
        t1 = t0 + NW
        dn0, dx0 = n_loads(t0, nb0, x0, si0, sa0)

        @pl.when(t1 < NU)
        def _():
            n_loads(t1, nb1, x1, si1, sa1)

        dn0.wait()
        dx0.wait()
        ds0 = pltpu.async_copy(x0, nacc_spm.at[nb0], ss0, add=True)
        histo(nb0, UN)

        @pl.when(t1 < NU)
        def _():
            base = pl.multiple_of(t1 * UN, UN)
            pltpu.make_async_copy(batch_hbm.at[pl.ds(base, UN)], nb1, si1).wait()
            pltpu.make_async_copy(x_hbm.at[pl.ds(base, UN)], x1, sa1).wait()
            ds1 = pltpu.async_copy(x1, nacc_spm.at[nb1], ss1, add=True)
            histo(nb1, UN)
            ds1.wait()

        ds0.wait()
        return carry

    lax.fori_loop(0, K_N // 2, node_pair, 0)

    cnt_out(ncnt_hbm)

    plsc.subcore_barrier()

    @pl.when(s == 0)
    def _():
        pltpu.sync_copy(eacc_spm, esum_hbm.at[c])
        pltpu.sync_copy(nacc_spm, nsum_hbm.at[c])


def _sc_segment_sums(edge_src, edge_attr, x, batch):
    mesh = plsc.VectorSubcoreMesh(core_axis_name="c", subcore_axis_name="s",
                                  num_cores=NC, num_subcores=NS)
    ze = jnp.zeros((B // NS, D_EDGE), jnp.float32)
    zn = jnp.zeros((B // NS, D_NODE), jnp.float32)
    zf = jnp.zeros((L * B,), jnp.float32)
    out_type = (
        jax.ShapeDtypeStruct((NC, B, D_EDGE), jnp.float32),
        jax.ShapeDtypeStruct((NC, B, D_NODE), jnp.float32),
        jax.ShapeDtypeStruct((NW, B), jnp.float32),
        jax.ShapeDtypeStruct((NW, B), jnp.float32),
    )
    scratch = [
        pltpu.VMEM_SHARED((N,), jnp.int32),
        pltpu.VMEM_SHARED((B, D_EDGE), jnp.float32),
        pltpu.VMEM_SHARED((B, D_NODE), jnp.float32),
        pltpu.VMEM((UE,), jnp.int32),
        pltpu.VMEM((UE,), jnp.int32),
        pltpu.VMEM((UE,), jnp.int32),
        pltpu.VMEM((UE,), jnp.int32),
        pltpu.VMEM((UE, D_EDGE), jnp.float32),
        pltpu.VMEM((UE, D_EDGE), jnp.float32),
        pltpu.VMEM((UN, D_NODE), jnp.float32),
        pltpu.VMEM((UN, D_NODE), jnp.float32),
        pltpu.VMEM((UN,), jnp.int32),
        pltpu.VMEM((UN,), jnp.int32),
        pltpu.VMEM((L * B,), jnp.float32),
        pltpu.VMEM((B,), jnp.float32),
        pltpu.SemaphoreType.DMA,
        pltpu.SemaphoreType.DMA,
        pltpu.SemaphoreType.DMA,
        pltpu.SemaphoreType.DMA,
        pltpu.SemaphoreType.DMA,
        pltpu.SemaphoreType.DMA,
        pltpu.SemaphoreType.DMA,
        pltpu.SemaphoreType.DMA,
    ]
    fn = pl.kernel(_sc_body, out_type=out_type, mesh=mesh,
                   scratch_types=scratch,
                   compiler_params=pltpu.CompilerParams(
                       needs_layout_passes=False,
                       use_tc_tiling_on_sc=False))
    return fn(edge_src, edge_attr, x, batch, ze, zn, zf)


def _tc_body(u_ref, esum_ref, nsum_ref, ecnt_ref, ncnt_ref,
             w1u_ref, w1e_ref, w1n_ref, b1_ref, w2_ref, b2_ref,
             gamma_ref, beta_ref, out_ref):
    e_sum = esum_ref[0] + esum_ref[1]
    n_sum = nsum_ref[0] + nsum_ref[1]
    e_cnt = jnp.sum(ecnt_ref[...], axis=0)
    n_cnt = jnp.sum(ncnt_ref[...], axis=0)
    e_mean = e_sum / jnp.maximum(e_cnt, 1.0)[:, None]
    n_mean = n_sum / jnp.maximum(n_cnt, 1.0)[:, None]
    u = u_ref[...]
    h = (jnp.dot(u, w1u_ref[...], preferred_element_type=jnp.float32)
         + jnp.dot(e_mean, w1e_ref[...], preferred_element_type=jnp.float32)
         + jnp.dot(n_mean, w1n_ref[...], preferred_element_type=jnp.float32)
         + b1_ref[...])
    h = jnp.maximum(h, 0.0)
    y = jnp.dot(h, w2_ref[...], preferred_element_type=jnp.float32) + b2_ref[...] + u
    mu = jnp.mean(y, axis=-1, keepdims=True)
    var = jnp.mean((y - mu) ** 2, axis=-1, keepdims=True)
    y = (y - mu) * lax.rsqrt(var + 1e-5)
    out_ref[...] = y * gamma_ref[...] + beta_ref[...]


def kernel(x, edge_index, edge_attr, u, batch, W1, b1, W2, b2, gamma, beta):
    edge_src = edge_index[0]
    esum, nsum, ecnt, ncnt = _sc_segment_sums(edge_src, edge_attr, x, batch)
    w1u = W1[:D_GLOB]
    w1e = W1[D_GLOB:D_GLOB + D_EDGE]
    w1n = W1[D_GLOB + D_EDGE:]
    return pl.pallas_call(
        _tc_body,
        out_shape=jax.ShapeDtypeStruct((B, D_GLOB), jnp.float32),
    )(u, esum, nsum, ecnt, ncnt, w1u, w1e, w1n,
      b1[None, :], W2, b2[None, :], gamma[None, :], beta[None, :])
